# baseline (speedup 1.0000x reference)
import functools

import jax
import jax.numpy as jnp
from jax.experimental import pallas as pl
from jax.experimental.pallas import tpu as pltpu
from jax.experimental.pallas import tpu_sc as plsc

MAX_SEQ_LEN = 32
EMBED_DIM = 128
SEQ_LEN = 20


def _build():
    mesh = plsc.ScalarSubcoreMesh(axis_name="c", num_cores=1)

    @functools.partial(
        pl.kernel,
        mesh=mesh,
        out_type=jax.ShapeDtypeStruct((MAX_SEQ_LEN, EMBED_DIM), jnp.float32),
        scratch_types=[
            pltpu.SMEM((SEQ_LEN,), jnp.int32),
            pltpu.SemaphoreType.DMA,
            pltpu.SemaphoreType.DMA,
        ],
    )
    def gather_kernel(x_hbm, w_hbm, out_hbm, idx_s, sem, idx_sem):
        idx_cp = pltpu.async_copy(x_hbm, idx_s, idx_sem)
        copies = []
        for i in range(SEQ_LEN, MAX_SEQ_LEN):
            copies.append(pltpu.async_copy(w_hbm.at[0], out_hbm.at[i], sem))
        idx_cp.wait()
        for i in range(SEQ_LEN):
            copies.append(
                pltpu.async_copy(w_hbm.at[idx_s[i]], out_hbm.at[i], sem)
            )
        for c in copies:
            c.wait()

    return gather_kernel


_GATHER = _build()


def kernel(x, key, weight):
    del key
    return _GATHER(x, weight)


# speedup vs baseline: 1.0163x; 1.0163x over previous
import functools

import jax
import jax.numpy as jnp
from jax.experimental import pallas as pl
from jax.experimental.pallas import tpu as pltpu
from jax.experimental.pallas import tpu_sc as plsc

MAX_SEQ_LEN = 32
EMBED_DIM = 128
SEQ_LEN = 20


def _build():
    mesh = plsc.ScalarSubcoreMesh(axis_name="c", num_cores=1)

    @functools.partial(
        pl.kernel,
        mesh=mesh,
        out_type=jax.ShapeDtypeStruct((MAX_SEQ_LEN, EMBED_DIM), jnp.float32),
        scratch_types=[
            pltpu.SMEM((SEQ_LEN,), jnp.int32),
            pltpu.SemaphoreType.DMA,
            pltpu.SemaphoreType.DMA,
        ],
    )
    def gather_kernel(x_hbm, w_hbm, out_hbm, idx_s, sem, idx_sem):
        idx_cp = pltpu.async_copy(x_hbm, idx_s, idx_sem)
        copies = []
        for i in range(SEQ_LEN, MAX_SEQ_LEN):
            copies.append(pltpu.async_copy(w_hbm.at[0], out_hbm.at[i], sem))
        idx_cp.wait()
        for i in range(SEQ_LEN):
            copies.append(
                pltpu.async_copy(w_hbm.at[idx_s[i]], out_hbm.at[i], sem)
            )
        del copies
        pltpu.make_async_copy(w_hbm, out_hbm, sem).wait()

    return gather_kernel


_GATHER = _build()


def kernel(x, key, weight):
    del key
    return _GATHER(x, weight)
